# Initial kernel scaffold; baseline (speedup 1.0000x reference)
import jax
import jax.numpy as jnp
from jax.experimental import pallas as pl


def kernel(obj_points, sr_points, hand_verts, hand_faces, face_normals):
    raise NotImplementedError("write your pallas kernel here")


# speedup vs baseline: 2.4360x; 2.4360x over previous
import jax
import jax.numpy as jnp
from jax.experimental import pallas as pl

N = 2048
_SB = 256
_FB = 256
_EPS = 1e-8
_BIG = 2**30


def _body(obj_t_ref, sr_ref, verts_t_ref, faces_ref, normals_ref,
          pen_ref, cmap_ref):
    ox = obj_t_ref[0:1, :]
    oy = obj_t_ref[1:2, :]
    oz = obj_t_ref[2:3, :]

    def nn_step(b, carry):
        m, idx = carry
        base = b * _SB
        sx = sr_ref[pl.ds(base, _SB), 0:1]
        sy = sr_ref[pl.ds(base, _SB), 1:2]
        sz = sr_ref[pl.ds(base, _SB), 2:3]
        ddx = ox - sx
        ddy = oy - sy
        ddz = oz - sz
        d2 = ddx * ddx + ddy * ddy + ddz * ddz
        bmin = jnp.min(d2, axis=0, keepdims=True)
        iota = jax.lax.broadcasted_iota(jnp.int32, (_SB, N), 0) + base
        barg = jnp.min(jnp.where(d2 == bmin, iota, _BIG), axis=0,
                       keepdims=True)
        take = bmin < m
        return jnp.where(take, bmin, m), jnp.where(take, barg, idx)

    m0 = jnp.full((1, N), jnp.inf, jnp.float32)
    i0 = jnp.zeros((1, N), jnp.int32)
    nn_d, nn_idx = jax.lax.fori_loop(0, N // _SB, nn_step, (m0, i0))

    cmap_ref[0:1, :] = 1.0 - 2.0 * (jax.nn.sigmoid(100.0 * nn_d) - 0.5)

    def trg_step(b, carry):
        tx, ty, tz = carry
        base = b * _SB
        iota = jax.lax.broadcasted_iota(jnp.int32, (_SB, N), 0) + base
        sel = iota == nn_idx
        sx = sr_ref[pl.ds(base, _SB), 0:1]
        sy = sr_ref[pl.ds(base, _SB), 1:2]
        sz = sr_ref[pl.ds(base, _SB), 2:3]
        tx = tx + jnp.sum(jnp.where(sel, sx, 0.0), axis=0, keepdims=True)
        ty = ty + jnp.sum(jnp.where(sel, sy, 0.0), axis=0, keepdims=True)
        tz = tz + jnp.sum(jnp.where(sel, sz, 0.0), axis=0, keepdims=True)
        return tx, ty, tz

    z0 = jnp.zeros((1, N), jnp.float32)
    tx, ty, tz = jax.lax.fori_loop(0, N // _SB, trg_step, (z0, z0, z0))

    dxr = tx - ox
    dyr = ty - oy
    dzr = tz - oz

    vx = verts_t_ref[0:1, :]
    vy = verts_t_ref[1:2, :]
    vz = verts_t_ref[2:3, :]

    def face_step(b, counts):
        base = b * _FB
        lane = jax.lax.broadcasted_iota(jnp.int32, (_FB, N), 1)

        def gather(col):
            fi = faces_ref[pl.ds(base, _FB), col:col + 1]
            sel = lane == fi
            gx = jnp.sum(jnp.where(sel, vx, 0.0), axis=1, keepdims=True)
            gy = jnp.sum(jnp.where(sel, vy, 0.0), axis=1, keepdims=True)
            gz = jnp.sum(jnp.where(sel, vz, 0.0), axis=1, keepdims=True)
            return gx, gy, gz

        v0x, v0y, v0z = gather(0)
        v1x, v1y, v1z = gather(1)
        v2x, v2y, v2z = gather(2)
        nx = normals_ref[pl.ds(base, _FB), 0:1]
        ny = normals_ref[pl.ds(base, _FB), 1:2]
        nz = normals_ref[pl.ds(base, _FB), 2:3]

        denom = nx * dxr + ny * dyr + nz * dzr
        valid = jnp.abs(denom) > _EPS
        safe_denom = jnp.where(valid, denom, 1.0)
        t = (nx * (v0x - ox) + ny * (v0y - oy) + nz * (v0z - oz)) / safe_denom
        px = ox + t * dxr
        py = oy + t * dyr
        pz = oz + t * dzr
        wx = px - v0x
        wy = py - v0y
        wz = pz - v0z
        e0x = v1x - v0x
        e0y = v1y - v0y
        e0z = v1z - v0z
        e1x = v2x - v0x
        e1y = v2y - v0y
        e1z = v2z - v0z
        dot00 = e0x * e0x + e0y * e0y + e0z * e0z
        dot01 = e0x * e1x + e0y * e1y + e0z * e1z
        dot11 = e1x * e1x + e1y * e1y + e1z * e1z
        dot0w = e0x * wx + e0y * wy + e0z * wz
        dot1w = e1x * wx + e1y * wy + e1z * wz
        den = dot00 * dot11 - dot01 * dot01
        safe_den = jnp.where(jnp.abs(den) > _EPS, den, 1.0)
        u = (dot11 * dot0w - dot01 * dot1w) / safe_den
        v = (dot00 * dot1w - dot01 * dot0w) / safe_den
        inside = (u >= -_EPS) & (v >= -_EPS) & (u + v <= 1.0 + _EPS)
        hit = valid & (t > _EPS) & inside
        return counts + jnp.sum(hit.astype(jnp.int32), axis=0, keepdims=True)

    counts = jax.lax.fori_loop(0, N // _FB, face_step,
                               jnp.zeros((1, N), jnp.int32))

    interior = (counts % 2) != 0
    pen2 = jnp.sum(jnp.where(interior, nn_d, 0.0), axis=1, keepdims=True)
    pen_ref[0:1, 0:1] = jnp.sqrt(pen2)


def kernel(obj_points, sr_points, hand_verts, hand_faces, face_normals):
    obj_t = obj_points.T
    verts_t = hand_verts.T
    faces = hand_faces.astype(jnp.int32)
    pen, cmap = pl.pallas_call(
        _body,
        out_shape=[
            jax.ShapeDtypeStruct((1, 1), jnp.float32),
            jax.ShapeDtypeStruct((1, N), jnp.float32),
        ],
    )(obj_t, sr_points, verts_t, faces, face_normals)
    return pen[0, 0], cmap[0]
